# baseline (speedup 1.0000x reference)
import functools

import jax
import jax.numpy as jnp
import ml_dtypes
import numpy as np
from jax import lax
from jax.experimental import pallas as pl
from jax.experimental.pallas import tpu as pltpu
from jax.experimental.pallas import tpu_sc as plsc

VOCAB_N = 50257
D = 1024
BATCH = 4
SEQ = 2048
NTOK = BATCH * SEQ
LN_EPS = 1e-05

NC = 2
NS = 16
NW = NC * NS
CH = 32

NSPLIT = 4
PB = BATCH // NSPLIT
PROWS = PB * SEQ
RPW = PROWS // NW
NCH = RPW // CH


def _sc_gather_part(table, idx2d, part):
    row_off = part * (PROWS // CH)
    mesh = plsc.VectorSubcoreMesh(core_axis_name="c", subcore_axis_name="s")

    @functools.partial(
        pl.kernel,
        mesh=mesh,
        out_type=jax.ShapeDtypeStruct((PROWS, D), jnp.float32),
        scratch_types=[
            pltpu.VMEM((NCH, CH), jnp.int32),
            pltpu.VMEM((CH, D), jnp.float32),
            pltpu.VMEM((CH, D), jnp.float32),
            pltpu.VMEM((CH, D), jnp.float32),
            pltpu.SemaphoreType.DMA,
            pltpu.SemaphoreType.DMA,
            pltpu.SemaphoreType.DMA,
            pltpu.SemaphoreType.DMA,
            pltpu.SemaphoreType.DMA,
            pltpu.SemaphoreType.DMA,
        ],
    )
    def k(table_hbm, idx_hbm, out_hbm, idx_v, b0, b1, b2,
          gs0, gs1, gs2, ws0, ws1, ws2):
        bufs = (b0, b1, b2)
        gsems = (gs0, gs1, gs2)
        wsems = (ws0, ws1, ws2)
        wid = lax.axis_index("s") * NC + lax.axis_index("c")
        pltpu.sync_copy(idx_hbm.at[pl.ds(row_off + wid * NCH, NCH)], idx_v)
        base = wid * RPW
        g = [None] * NCH
        w = [None] * NCH
        for c in range(min(3, NCH)):
            g[c] = pltpu.async_copy(
                table_hbm.at[idx_v.at[c]], bufs[c % 3], gsems[c % 3])
        for c in range(NCH):
            k3 = c % 3
            g[c].wait()
            w[c] = pltpu.async_copy(
                bufs[k3], out_hbm.at[pl.ds(base + c * CH, CH)], wsems[k3])
            nxt = c + 3
            if nxt < NCH:
                w[c].wait()
                g[nxt] = pltpu.async_copy(
                    table_hbm.at[idx_v.at[nxt]], bufs[k3], gsems[k3])
        for c in range(max(0, NCH - 3), NCH):
            w[c].wait()

    return k(table, idx2d)


SB = 512

_FREQ_HALF = np.arange(SEQ, dtype=np.float32)[:, None] * (
    1.0 / (10000.0 ** (np.arange(0, D, 2, dtype=np.float32) / np.float32(D)))
)[None, :]
_COS_HALF = np.cos(_FREQ_HALF).astype(ml_dtypes.bfloat16)
_SIN_HALF = np.sin(_FREQ_HALF).astype(ml_dtypes.bfloat16)


def _tc_body(emb_ref, cos_ref, sin_ref, w_ref, b_ref, out_ref):
    cos_f = cos_ref[...].astype(jnp.float32)
    sin_f = sin_ref[...].astype(jnp.float32)
    cos_v = jnp.concatenate([cos_f, cos_f], axis=-1)
    sin_v = jnp.concatenate([sin_f, sin_f], axis=-1)
    x = emb_ref[...]
    xr = pltpu.roll(x, 1, 2)
    y = x * cos_v[None] + xr * sin_v[None]
    mu = jnp.mean(y, axis=-1, keepdims=True)
    yc = y - mu
    var = jnp.mean(yc * yc, axis=-1, keepdims=True)
    out = yc * lax.rsqrt(var + LN_EPS) * w_ref[...] + b_ref[...]
    out_ref[0] = out


def _tc_body_alias(prev_ref, emb_ref, cos_ref, sin_ref, w_ref, b_ref, out_ref):
    del prev_ref
    _tc_body(emb_ref, cos_ref, sin_ref, w_ref, b_ref, out_ref)


def _tc_rotary_ln_part(emb3, cos_h, sin_h, lnw, lnb, part, prev=None,
                       interpret=False):
    common = dict(
        grid=(SEQ // SB,),
        out_specs=pl.BlockSpec(
            (1, PB, SB, D), lambda i: (0, part, i, 0)),
        out_shape=jax.ShapeDtypeStruct((1, BATCH, SEQ, D), jnp.float32),
        interpret=interpret,
    )
    data_specs = [
        pl.BlockSpec((PB, SB, D), lambda i: (0, i, 0)),
        pl.BlockSpec((SB, D // 2), lambda i: (i, 0)),
        pl.BlockSpec((SB, D // 2), lambda i: (i, 0)),
        pl.BlockSpec((1, D), lambda i: (0, 0)),
        pl.BlockSpec((1, D), lambda i: (0, 0)),
    ]
    if prev is None:
        return pl.pallas_call(_tc_body, in_specs=data_specs, **common)(
            emb3, cos_h, sin_h, lnw, lnb)
    prev_spec = pl.BlockSpec(memory_space=pl.ANY)
    return pl.pallas_call(
        _tc_body_alias,
        in_specs=[prev_spec] + data_specs,
        input_output_aliases={0: 0},
        **common,
    )(prev, emb3, cos_h, sin_h, lnw, lnb)


def kernel(input_ids, token_emb, ln_w, ln_b):
    ids = input_ids.reshape(-1).astype(jnp.int32)
    cos_h = jnp.asarray(_COS_HALF)
    sin_h = jnp.asarray(_SIN_HALF)
    lnw = ln_w.reshape(1, D)
    lnb = ln_b.reshape(1, D)
    idx2d = ids.reshape(NTOK // CH, CH)
    embs = [_sc_gather_part(token_emb, idx2d, q) for q in range(NSPLIT)]
    out = None
    for q in range(NSPLIT):
        out = _tc_rotary_ln_part(
            embs[q].reshape(PB, SEQ, D), cos_h, sin_h, lnw, lnb,
            part=q, prev=out)
    return out


# speedup vs baseline: 1.0735x; 1.0735x over previous
import functools

import jax
import jax.numpy as jnp
import ml_dtypes
import numpy as np
from jax import lax
from jax.experimental import pallas as pl
from jax.experimental.pallas import tpu as pltpu
from jax.experimental.pallas import tpu_sc as plsc

VOCAB_N = 50257
D = 1024
BATCH = 4
SEQ = 2048
NTOK = BATCH * SEQ
LN_EPS = 1e-05

NC = 2
NS = 16
NW = NC * NS
CH = 32

NSPLIT = 2
PB = BATCH // NSPLIT
PROWS = PB * SEQ
RPW = PROWS // NW
NCH = RPW // CH


def _sc_gather_part(table, idx2d, part):
    row_off = part * (PROWS // CH)
    mesh = plsc.VectorSubcoreMesh(core_axis_name="c", subcore_axis_name="s")

    @functools.partial(
        pl.kernel,
        mesh=mesh,
        out_type=jax.ShapeDtypeStruct((PROWS, D), jnp.float32),
        scratch_types=[
            pltpu.VMEM((NCH, CH), jnp.int32),
            pltpu.VMEM((CH, D), jnp.float32),
            pltpu.VMEM((CH, D), jnp.float32),
            pltpu.VMEM((CH, D), jnp.float32),
            pltpu.SemaphoreType.DMA,
            pltpu.SemaphoreType.DMA,
            pltpu.SemaphoreType.DMA,
            pltpu.SemaphoreType.DMA,
            pltpu.SemaphoreType.DMA,
            pltpu.SemaphoreType.DMA,
        ],
    )
    def k(table_hbm, idx_hbm, out_hbm, idx_v, b0, b1, b2,
          gs0, gs1, gs2, ws0, ws1, ws2):
        bufs = (b0, b1, b2)
        gsems = (gs0, gs1, gs2)
        wsems = (ws0, ws1, ws2)
        wid = lax.axis_index("s") * NC + lax.axis_index("c")
        pltpu.sync_copy(idx_hbm.at[pl.ds(row_off + wid * NCH, NCH)], idx_v)
        base = wid * RPW
        g = [None] * NCH
        w = [None] * NCH
        for c in range(min(3, NCH)):
            g[c] = pltpu.async_copy(
                table_hbm.at[idx_v.at[c]], bufs[c % 3], gsems[c % 3])
        for c in range(NCH):
            k3 = c % 3
            g[c].wait()
            w[c] = pltpu.async_copy(
                bufs[k3], out_hbm.at[pl.ds(base + c * CH, CH)], wsems[k3])
            nxt = c + 3
            if nxt < NCH:
                w[c].wait()
                g[nxt] = pltpu.async_copy(
                    table_hbm.at[idx_v.at[nxt]], bufs[k3], gsems[k3])
        for c in range(max(0, NCH - 3), NCH):
            w[c].wait()

    return k(table, idx2d)


SB = 512

_FREQ_HALF = np.arange(SEQ, dtype=np.float32)[:, None] * (
    1.0 / (10000.0 ** (np.arange(0, D, 2, dtype=np.float32) / np.float32(D)))
)[None, :]
_COS_FULL = np.concatenate([np.cos(_FREQ_HALF)] * 2, axis=-1).astype(
    ml_dtypes.bfloat16)
_SIN_FULL = np.concatenate([np.sin(_FREQ_HALF)] * 2, axis=-1).astype(
    ml_dtypes.bfloat16)


def _tc_body(emb_ref, cos_ref, sin_ref, w_ref, b_ref, out_ref):
    cos_v = cos_ref[...].astype(jnp.float32)
    sin_v = sin_ref[...].astype(jnp.float32)
    x = emb_ref[...]
    xr = pltpu.roll(x, 1, 2)
    y = x * cos_v[None] + xr * sin_v[None]
    mu = jnp.mean(y, axis=-1, keepdims=True)
    var = jnp.mean(y * y, axis=-1, keepdims=True) - mu * mu
    out = (y - mu) * lax.rsqrt(var + LN_EPS) * w_ref[...] + b_ref[...]
    out_ref[0] = out


def _tc_body_alias(prev_ref, emb_ref, cos_ref, sin_ref, w_ref, b_ref, out_ref):
    del prev_ref
    _tc_body(emb_ref, cos_ref, sin_ref, w_ref, b_ref, out_ref)


def _tc_rotary_ln_part(emb3, cos_h, sin_h, lnw, lnb, part, prev=None,
                       interpret=False):
    common = dict(
        grid=(SEQ // SB,),
        out_specs=pl.BlockSpec(
            (1, PB, SB, D), lambda i: (0, part, i, 0)),
        out_shape=jax.ShapeDtypeStruct((1, BATCH, SEQ, D), jnp.float32),
        interpret=interpret,
    )
    data_specs = [
        pl.BlockSpec((PB, SB, D), lambda i: (0, i, 0)),
        pl.BlockSpec((SB, D), lambda i: (i, 0)),
        pl.BlockSpec((SB, D), lambda i: (i, 0)),
        pl.BlockSpec((1, D), lambda i: (0, 0)),
        pl.BlockSpec((1, D), lambda i: (0, 0)),
    ]
    if prev is None:
        return pl.pallas_call(_tc_body, in_specs=data_specs, **common)(
            emb3, cos_h, sin_h, lnw, lnb)
    prev_spec = pl.BlockSpec(memory_space=pl.ANY)
    return pl.pallas_call(
        _tc_body_alias,
        in_specs=[prev_spec] + data_specs,
        input_output_aliases={0: 0},
        **common,
    )(prev, emb3, cos_h, sin_h, lnw, lnb)


def kernel(input_ids, token_emb, ln_w, ln_b):
    ids = input_ids.reshape(-1).astype(jnp.int32)
    cos_h = jnp.asarray(_COS_FULL)
    sin_h = jnp.asarray(_SIN_FULL)
    lnw = ln_w.reshape(1, D)
    lnb = ln_b.reshape(1, D)
    idx2d = ids.reshape(NTOK // CH, CH)
    embs = [_sc_gather_part(token_emb, idx2d, q) for q in range(NSPLIT)]
    out = None
    for q in range(NSPLIT):
        out = _tc_rotary_ln_part(
            embs[q].reshape(PB, SEQ, D), cos_h, sin_h, lnw, lnb,
            part=q, prev=out)
    return out


# speedup vs baseline: 1.0906x; 1.0159x over previous
import functools

import jax
import jax.numpy as jnp
import ml_dtypes
import numpy as np
from jax import lax
from jax.experimental import pallas as pl
from jax.experimental.pallas import tpu as pltpu
from jax.experimental.pallas import tpu_sc as plsc

VOCAB_N = 50257
D = 1024
BATCH = 4
SEQ = 2048
NTOK = BATCH * SEQ
LN_EPS = 1e-05

NC = 2
NS = 16
NW = NC * NS
CH = 32

NSPLIT = 2
PB = BATCH // NSPLIT
PROWS = PB * SEQ
RPW = PROWS // NW
NCH = RPW // CH


def _sc_gather_part(table, idx2d, part):
    row_off = part * (PROWS // CH)
    mesh = plsc.VectorSubcoreMesh(core_axis_name="c", subcore_axis_name="s")

    @functools.partial(
        pl.kernel,
        mesh=mesh,
        out_type=jax.ShapeDtypeStruct((PROWS, D), jnp.float32),
        scratch_types=[
            pltpu.VMEM((NCH, CH), jnp.int32),
            pltpu.VMEM((CH, D), jnp.float32),
            pltpu.VMEM((CH, D), jnp.float32),
            pltpu.VMEM((CH, D), jnp.float32),
            pltpu.SemaphoreType.DMA,
            pltpu.SemaphoreType.DMA,
            pltpu.SemaphoreType.DMA,
            pltpu.SemaphoreType.DMA,
            pltpu.SemaphoreType.DMA,
            pltpu.SemaphoreType.DMA,
        ],
    )
    def k(table_hbm, idx_hbm, out_hbm, idx_v, b0, b1, b2,
          gs0, gs1, gs2, ws0, ws1, ws2):
        bufs = (b0, b1, b2)
        gsems = (gs0, gs1, gs2)
        wsems = (ws0, ws1, ws2)
        wid = lax.axis_index("s") * NC + lax.axis_index("c")
        pltpu.sync_copy(idx_hbm.at[pl.ds(row_off + wid * NCH, NCH)], idx_v)
        base = wid * RPW
        g = [None] * NCH
        w = [None] * NCH
        for c in range(min(3, NCH)):
            g[c] = pltpu.async_copy(
                table_hbm.at[idx_v.at[c]], bufs[c % 3], gsems[c % 3])
        for c in range(NCH):
            k3 = c % 3
            g[c].wait()
            w[c] = pltpu.async_copy(
                bufs[k3], out_hbm.at[pl.ds(base + c * CH, CH)], wsems[k3])
            nxt = c + 3
            if nxt < NCH:
                w[c].wait()
                g[nxt] = pltpu.async_copy(
                    table_hbm.at[idx_v.at[nxt]], bufs[k3], gsems[k3])
        for c in range(max(0, NCH - 3), NCH):
            w[c].wait()

    return k(table, idx2d)


SB = 512

_FREQ_HALF = np.arange(SEQ, dtype=np.float32)[:, None] * (
    1.0 / (10000.0 ** (np.arange(0, D, 2, dtype=np.float32) / np.float32(D)))
)[None, :]
_COS_HALF = np.cos(_FREQ_HALF).astype(ml_dtypes.bfloat16)
_SIN_HALF = np.sin(_FREQ_HALF).astype(ml_dtypes.bfloat16)


def _tc_body(emb_ref, cos_ref, sin_ref, w_ref, b_ref, out_ref):
    cos_f = cos_ref[...].astype(jnp.float32)
    sin_f = sin_ref[...].astype(jnp.float32)
    cos_v = jnp.concatenate([cos_f, cos_f], axis=-1)
    sin_v = jnp.concatenate([sin_f, sin_f], axis=-1)
    x = emb_ref[...]
    xr = pltpu.roll(x, 1, 2)
    y = x * cos_v[None] + xr * sin_v[None]
    mu = jnp.mean(y, axis=-1, keepdims=True)
    var = jnp.mean(y * y, axis=-1, keepdims=True) - mu * mu
    out = (y - mu) * lax.rsqrt(var + LN_EPS) * w_ref[...] + b_ref[...]
    out_ref[0] = out


def _tc_body_alias(prev_ref, emb_ref, cos_ref, sin_ref, w_ref, b_ref, out_ref):
    del prev_ref
    _tc_body(emb_ref, cos_ref, sin_ref, w_ref, b_ref, out_ref)


def _tc_rotary_ln_part(emb3, cos_h, sin_h, lnw, lnb, part, prev=None,
                       interpret=False):
    common = dict(
        grid=(SEQ // SB,),
        out_specs=pl.BlockSpec(
            (1, PB, SB, D), lambda i: (0, part, i, 0)),
        out_shape=jax.ShapeDtypeStruct((1, BATCH, SEQ, D), jnp.float32),
        interpret=interpret,
    )
    data_specs = [
        pl.BlockSpec((PB, SB, D), lambda i: (0, i, 0)),
        pl.BlockSpec((SB, D // 2), lambda i: (i, 0)),
        pl.BlockSpec((SB, D // 2), lambda i: (i, 0)),
        pl.BlockSpec((1, D), lambda i: (0, 0)),
        pl.BlockSpec((1, D), lambda i: (0, 0)),
    ]
    if prev is None:
        return pl.pallas_call(_tc_body, in_specs=data_specs, **common)(
            emb3, cos_h, sin_h, lnw, lnb)
    prev_spec = pl.BlockSpec(memory_space=pl.ANY)
    return pl.pallas_call(
        _tc_body_alias,
        in_specs=[prev_spec] + data_specs,
        input_output_aliases={0: 0},
        **common,
    )(prev, emb3, cos_h, sin_h, lnw, lnb)


def kernel(input_ids, token_emb, ln_w, ln_b):
    ids = input_ids.reshape(-1).astype(jnp.int32)
    cos_h = jnp.asarray(_COS_HALF)
    sin_h = jnp.asarray(_SIN_HALF)
    lnw = ln_w.reshape(1, D)
    lnb = ln_b.reshape(1, D)
    idx2d = ids.reshape(NTOK // CH, CH)
    embs = [_sc_gather_part(token_emb, idx2d, q) for q in range(NSPLIT)]
    out = None
    for q in range(NSPLIT):
        out = _tc_rotary_ln_part(
            embs[q].reshape(PB, SEQ, D), cos_h, sin_h, lnw, lnb,
            part=q, prev=out)
    return out


# speedup vs baseline: 1.0928x; 1.0020x over previous
import functools

import jax
import jax.numpy as jnp
import ml_dtypes
import numpy as np
from jax import lax
from jax.experimental import pallas as pl
from jax.experimental.pallas import tpu as pltpu
from jax.experimental.pallas import tpu_sc as plsc

VOCAB_N = 50257
D = 1024
BATCH = 4
SEQ = 2048
NTOK = BATCH * SEQ
LN_EPS = 1e-05

NC = 2
NS = 16
NW = NC * NS
CH = 32

NSPLIT = 2
PB = BATCH // NSPLIT
PROWS = PB * SEQ
RPW = PROWS // NW
NCH = RPW // CH


def _sc_gather_part(table, ids2d, part):
    mesh = plsc.VectorSubcoreMesh(core_axis_name="c", subcore_axis_name="s")

    @functools.partial(
        pl.kernel,
        mesh=mesh,
        out_type=jax.ShapeDtypeStruct((PROWS, D), jnp.float32),
        scratch_types=[
            pltpu.VMEM((1, RPW), jnp.int32),
            pltpu.VMEM((CH, D), jnp.float32),
            pltpu.VMEM((CH, D), jnp.float32),
            pltpu.VMEM((CH, D), jnp.float32),
            pltpu.SemaphoreType.DMA,
            pltpu.SemaphoreType.DMA,
            pltpu.SemaphoreType.DMA,
            pltpu.SemaphoreType.DMA,
            pltpu.SemaphoreType.DMA,
            pltpu.SemaphoreType.DMA,
        ],
    )
    def k(table_hbm, idx_hbm, out_hbm, idx_v, b0, b1, b2,
          gs0, gs1, gs2, ws0, ws1, ws2):
        bufs = (b0, b1, b2)
        gsems = (gs0, gs1, gs2)
        wsems = (ws0, ws1, ws2)
        wid = lax.axis_index("s") * NC + lax.axis_index("c")
        bat = part * PB + wid // (SEQ // RPW)
        col = (wid * RPW) % SEQ
        pltpu.sync_copy(
            idx_hbm.at[pl.ds(bat, 1), pl.ds(col, RPW)], idx_v)
        base = wid * RPW
        g = [None] * NCH
        w = [None] * NCH
        for c in range(min(3, NCH)):
            g[c] = pltpu.async_copy(
                table_hbm.at[idx_v.at[0, pl.ds(c * CH, CH)]],
                bufs[c % 3], gsems[c % 3])
        for c in range(NCH):
            k3 = c % 3
            g[c].wait()
            w[c] = pltpu.async_copy(
                bufs[k3], out_hbm.at[pl.ds(base + c * CH, CH)], wsems[k3])
            nxt = c + 3
            if nxt < NCH:
                w[c].wait()
                g[nxt] = pltpu.async_copy(
                    table_hbm.at[idx_v.at[0, pl.ds(nxt * CH, CH)]],
                    bufs[k3], gsems[k3])
        for c in range(max(0, NCH - 3), NCH):
            w[c].wait()

    return k(table, ids2d)


SB = 512

_FREQ_HALF = np.arange(SEQ, dtype=np.float32)[:, None] * (
    1.0 / (10000.0 ** (np.arange(0, D, 2, dtype=np.float32) / np.float32(D)))
)[None, :]
_COS_HALF = np.cos(_FREQ_HALF).astype(ml_dtypes.bfloat16)
_SIN_HALF = np.sin(_FREQ_HALF).astype(ml_dtypes.bfloat16)


def _tc_body(emb_ref, cos_ref, sin_ref, w_ref, b_ref, out_ref):
    cos_f = cos_ref[...].astype(jnp.float32)
    sin_f = sin_ref[...].astype(jnp.float32)
    cos_v = jnp.concatenate([cos_f, cos_f], axis=-1)
    sin_v = jnp.concatenate([sin_f, sin_f], axis=-1)
    x = emb_ref[...]
    xr = pltpu.roll(x, 1, 2)
    y = x * cos_v[None] + xr * sin_v[None]
    mu = jnp.mean(y, axis=-1, keepdims=True)
    var = jnp.mean(y * y, axis=-1, keepdims=True) - mu * mu
    out = (y - mu) * lax.rsqrt(var + LN_EPS) * w_ref[...] + b_ref[...]
    out_ref[0] = out


def _tc_body_alias(prev_ref, emb_ref, cos_ref, sin_ref, w_ref, b_ref, out_ref):
    del prev_ref
    _tc_body(emb_ref, cos_ref, sin_ref, w_ref, b_ref, out_ref)


def _tc_rotary_ln_part(emb3, cos_h, sin_h, lnw, lnb, part, prev=None,
                       interpret=False):
    common = dict(
        grid=(SEQ // SB,),
        out_specs=pl.BlockSpec(
            (1, PB, SB, D), lambda i: (0, part, i, 0)),
        out_shape=jax.ShapeDtypeStruct((1, BATCH, SEQ, D), jnp.float32),
        interpret=interpret,
    )
    data_specs = [
        pl.BlockSpec((PB, SB, D), lambda i: (0, i, 0)),
        pl.BlockSpec((SB, D // 2), lambda i: (i, 0)),
        pl.BlockSpec((SB, D // 2), lambda i: (i, 0)),
        pl.BlockSpec((1, D), lambda i: (0, 0)),
        pl.BlockSpec((1, D), lambda i: (0, 0)),
    ]
    if prev is None:
        return pl.pallas_call(_tc_body, in_specs=data_specs, **common)(
            emb3, cos_h, sin_h, lnw, lnb)
    prev_spec = pl.BlockSpec(memory_space=pl.ANY)
    return pl.pallas_call(
        _tc_body_alias,
        in_specs=[prev_spec] + data_specs,
        input_output_aliases={0: 0},
        **common,
    )(prev, emb3, cos_h, sin_h, lnw, lnb)


def kernel(input_ids, token_emb, ln_w, ln_b):
    ids2d = input_ids.astype(jnp.int32)
    cos_h = jnp.asarray(_COS_HALF)
    sin_h = jnp.asarray(_SIN_HALF)
    lnw = ln_w.reshape(1, D)
    lnb = ln_b.reshape(1, D)
    embs = [_sc_gather_part(token_emb, ids2d, q) for q in range(NSPLIT)]
    out = None
    for q in range(NSPLIT):
        out = _tc_rotary_ln_part(
            embs[q].reshape(PB, SEQ, D), cos_h, sin_h, lnw, lnb,
            part=q, prev=out)
    return out


# speedup vs baseline: 1.1052x; 1.0114x over previous
import functools

import jax
import jax.numpy as jnp
import ml_dtypes
import numpy as np
from jax import lax
from jax.experimental import pallas as pl
from jax.experimental.pallas import tpu as pltpu
from jax.experimental.pallas import tpu_sc as plsc

VOCAB_N = 50257
D = 1024
BATCH = 4
SEQ = 2048
NTOK = BATCH * SEQ
LN_EPS = 1e-05

NC = 2
NS = 16
NW = NC * NS
CH = 32

NSPLIT = 2
PB = BATCH // NSPLIT
PROWS = PB * SEQ
RPW = PROWS // NW
NCH = RPW // CH


def _sc_gather_part(table, ids2d, part):
    mesh = plsc.VectorSubcoreMesh(core_axis_name="c", subcore_axis_name="s")

    @functools.partial(
        pl.kernel,
        mesh=mesh,
        out_type=jax.ShapeDtypeStruct((PROWS, D), jnp.float32),
        scratch_types=[
            pltpu.VMEM((1, RPW), jnp.int32),
            pltpu.VMEM((CH, D), jnp.float32),
            pltpu.VMEM((CH, D), jnp.float32),
            pltpu.VMEM((CH, D), jnp.float32),
            pltpu.SemaphoreType.DMA,
            pltpu.SemaphoreType.DMA,
            pltpu.SemaphoreType.DMA,
            pltpu.SemaphoreType.DMA,
            pltpu.SemaphoreType.DMA,
            pltpu.SemaphoreType.DMA,
        ],
    )
    def k(table_hbm, idx_hbm, out_hbm, idx_v, b0, b1, b2,
          gs0, gs1, gs2, ws0, ws1, ws2):
        bufs = (b0, b1, b2)
        gsems = (gs0, gs1, gs2)
        wsems = (ws0, ws1, ws2)
        wid = lax.axis_index("s") * NC + lax.axis_index("c")
        bat = part * PB + wid // (SEQ // RPW)
        col = (wid * RPW) % SEQ
        pltpu.sync_copy(
            idx_hbm.at[pl.ds(bat, 1), pl.ds(col, RPW)], idx_v)
        base = wid * RPW
        g = [None] * NCH
        w = [None] * NCH
        for c in range(min(3, NCH)):
            g[c] = pltpu.async_copy(
                table_hbm.at[idx_v.at[0, pl.ds(c * CH, CH)]],
                bufs[c % 3], gsems[c % 3])
        for c in range(NCH):
            k3 = c % 3
            g[c].wait()
            w[c] = pltpu.async_copy(
                bufs[k3], out_hbm.at[pl.ds(base + c * CH, CH)], wsems[k3])
            nxt = c + 3
            if nxt < NCH:
                w[c].wait()
                g[nxt] = pltpu.async_copy(
                    table_hbm.at[idx_v.at[0, pl.ds(nxt * CH, CH)]],
                    bufs[k3], gsems[k3])
        for c in range(max(0, NCH - 3), NCH):
            w[c].wait()

    return k(table, ids2d)


SB = 1024

_FREQ_HALF = np.arange(SEQ, dtype=np.float32)[:, None] * (
    1.0 / (10000.0 ** (np.arange(0, D, 2, dtype=np.float32) / np.float32(D)))
)[None, :]
_COS_HALF = np.cos(_FREQ_HALF).astype(ml_dtypes.bfloat16)
_SIN_HALF = np.sin(_FREQ_HALF).astype(ml_dtypes.bfloat16)


def _tc_body(emb_ref, cos_ref, sin_ref, w_ref, b_ref, out_ref):
    cos_f = cos_ref[...].astype(jnp.float32)
    sin_f = sin_ref[...].astype(jnp.float32)
    cos_v = jnp.concatenate([cos_f, cos_f], axis=-1)
    sin_v = jnp.concatenate([sin_f, sin_f], axis=-1)
    x = emb_ref[...]
    xr = pltpu.roll(x, 1, 2)
    y = x * cos_v[None] + xr * sin_v[None]
    mu = jnp.mean(y, axis=-1, keepdims=True)
    var = jnp.mean(y * y, axis=-1, keepdims=True) - mu * mu
    out = (y - mu) * lax.rsqrt(var + LN_EPS) * w_ref[...] + b_ref[...]
    out_ref[0] = out


def _tc_body_alias(prev_ref, emb_ref, cos_ref, sin_ref, w_ref, b_ref, out_ref):
    del prev_ref
    _tc_body(emb_ref, cos_ref, sin_ref, w_ref, b_ref, out_ref)


def _tc_rotary_ln_part(emb3, cos_h, sin_h, lnw, lnb, part, prev=None,
                       interpret=False):
    common = dict(
        grid=(SEQ // SB,),
        out_specs=pl.BlockSpec(
            (1, PB, SB, D), lambda i: (0, part, i, 0)),
        out_shape=jax.ShapeDtypeStruct((1, BATCH, SEQ, D), jnp.float32),
        interpret=interpret,
    )
    data_specs = [
        pl.BlockSpec((PB, SB, D), lambda i: (0, i, 0)),
        pl.BlockSpec((SB, D // 2), lambda i: (i, 0)),
        pl.BlockSpec((SB, D // 2), lambda i: (i, 0)),
        pl.BlockSpec((1, D), lambda i: (0, 0)),
        pl.BlockSpec((1, D), lambda i: (0, 0)),
    ]
    if prev is None:
        return pl.pallas_call(_tc_body, in_specs=data_specs, **common)(
            emb3, cos_h, sin_h, lnw, lnb)
    prev_spec = pl.BlockSpec(memory_space=pl.ANY)
    return pl.pallas_call(
        _tc_body_alias,
        in_specs=[prev_spec] + data_specs,
        input_output_aliases={0: 0},
        **common,
    )(prev, emb3, cos_h, sin_h, lnw, lnb)


def kernel(input_ids, token_emb, ln_w, ln_b):
    ids2d = input_ids.astype(jnp.int32)
    cos_h = jnp.asarray(_COS_HALF)
    sin_h = jnp.asarray(_SIN_HALF)
    lnw = ln_w.reshape(1, D)
    lnb = ln_b.reshape(1, D)
    embs = [_sc_gather_part(token_emb, ids2d, q) for q in range(NSPLIT)]
    out = None
    for q in range(NSPLIT):
        out = _tc_rotary_ln_part(
            embs[q].reshape(PB, SEQ, D), cos_h, sin_h, lnw, lnb,
            part=q, prev=out)
    return out
